# baseline (speedup 1.0000x reference)
import functools

import jax
import jax.numpy as jnp
from jax import lax
from jax.experimental import pallas as pl
from jax.experimental.pallas import tpu as pltpu
from jax.experimental.pallas import tpu_sc as plsc

K = 8192
D = 256
N = 8192
BETA = 0.25
BN = 256
NB = N // BN


def _tc_body(sx_ref, sw_ref, x_ref, w_ref,
             idx_ref, enc_ref, scal_ref,
             hist_ref, acc_ref):
    i = pl.program_id(0)

    @pl.when(i == 0)
    def _init():
        hist_ref[...] = jnp.zeros_like(hist_ref)
        acc_ref[0, 0] = 0.0

    x2 = x_ref[...]
    w = w_ref[...]
    mm = lax.dot_general(x2, w, (((1,), (1,)), ((), ())),
                         preferred_element_type=jnp.float32)
    t = sx_ref[...] + sw_ref[...]
    d = t + mm
    dmin = jnp.min(d, axis=1, keepdims=True)
    iota = lax.broadcasted_iota(jnp.int32, (BN, K), 1)
    idx = jnp.min(jnp.where(d == dmin, iota, K), axis=1)
    idx_ref[...] = idx[:, None]
    enc = (iota == idx[:, None]).astype(jnp.float32)
    enc_ref[...] = enc
    hist_ref[...] += jnp.sum(enc, axis=0, keepdims=True)
    acc_ref[0, 0] += jnp.sum(dmin)

    @pl.when(i == NB - 1)
    def _fin():
        p = hist_ref[...] * (1.0 / N)
        s = jnp.sum(p * jnp.log(p + 1e-10))
        scal_ref[0, 0] = (1.0 + BETA) * (acc_ref[0, 0] / (N * D))
        scal_ref[0, 1] = jnp.exp(-s)


_tc_call = pl.pallas_call(
    _tc_body,
    grid=(NB,),
    in_specs=[
        pl.BlockSpec((BN, 1), lambda i: (i, 0)),
        pl.BlockSpec((1, K), lambda i: (0, 0)),
        pl.BlockSpec((BN, D), lambda i: (i, 0)),
        pl.BlockSpec((K, D), lambda i: (0, 0)),
    ],
    out_specs=[
        pl.BlockSpec((BN, 1), lambda i: (i, 0)),
        pl.BlockSpec((BN, K), lambda i: (i, 0)),
        pl.BlockSpec(memory_space=pltpu.SMEM),
    ],
    out_shape=[
        jax.ShapeDtypeStruct((N, 1), jnp.int32),
        jax.ShapeDtypeStruct((N, K), jnp.float32),
        jax.ShapeDtypeStruct((1, 2), jnp.float32),
    ],
    scratch_shapes=[
        pltpu.VMEM((1, K), jnp.float32),
        pltpu.SMEM((1, 1), jnp.float32),
    ],
    compiler_params=pltpu.CompilerParams(
        dimension_semantics=("arbitrary",),
    ),
)


def _make_sc_gather():
    info = plsc.get_sparse_core_info()
    nc, ns = info.num_cores, info.num_subcores
    nw = nc * ns
    b_per_w = N // nw
    mesh = plsc.VectorSubcoreMesh(core_axis_name="c", subcore_axis_name="s")

    @functools.partial(
        pl.kernel, mesh=mesh,
        out_type=jax.ShapeDtypeStruct((N, D), jnp.float32),
        scratch_types=[
            pltpu.VMEM((b_per_w,), jnp.int32),
            pltpu.VMEM((b_per_w, D), jnp.float32),
            pltpu.SemaphoreType.DMA,
        ],
    )
    def gather_k(w_hbm, idx_hbm, out_hbm, idx_v, rows_v, sem):
        wid = lax.axis_index("s") * nc + lax.axis_index("c")
        base = wid * b_per_w
        pltpu.sync_copy(idx_hbm.at[pl.ds(base, b_per_w)], idx_v)
        pltpu.async_copy(w_hbm.at[idx_v], rows_v, sem).wait()
        pltpu.sync_copy(rows_v, out_hbm.at[pl.ds(base, b_per_w)])

    return gather_k


def kernel(inputs, W):
    x4 = jnp.transpose(inputs, (0, 2, 3, 1))
    input_shape = x4.shape
    flat = x4.reshape(-1, D)
    sx = jnp.sum(flat ** 2, axis=1, keepdims=True)
    sw = jnp.sum(W ** 2, axis=1).reshape(1, K)
    idx2, encodings, scal = _tc_call(sx, sw, -2.0 * flat, W)
    idx = idx2.reshape(N)
    qflat = _make_sc_gather()(W, idx)
    quantized_st = flat + (qflat - flat)
    quantized_st = jnp.transpose(quantized_st.reshape(input_shape), (0, 3, 1, 2))
    loss = scal[0, 0]
    perplexity = scal[0, 1]
    return (loss, quantized_st, perplexity, encodings)


# speedup vs baseline: 1.0784x; 1.0784x over previous
import functools

import jax
import jax.numpy as jnp
from jax import lax
from jax.experimental import pallas as pl
from jax.experimental.pallas import tpu as pltpu
from jax.experimental.pallas import tpu_sc as plsc

K = 8192
D = 256
N = 8192
BETA = 0.25
BN = 256
NB = N // BN


def _tc_body(sx_ref, sw_ref, x_ref, w_ref,
             idx_ref, enc_ref, scal_ref,
             hist_ref, acc_ref):
    i = pl.program_id(0)

    @pl.when(i == 0)
    def _init():
        hist_ref[...] = jnp.zeros_like(hist_ref)
        acc_ref[0, 0] = 0.0

    x = x_ref[...]
    w = w_ref[...]
    mm = lax.dot_general(x, w, (((1,), (1,)), ((), ())),
                         preferred_element_type=jnp.float32)
    t = sx_ref[...] + sw_ref[...]
    d = t - 2.0 * mm
    dmin = jnp.min(d, axis=1, keepdims=True)
    iota = lax.broadcasted_iota(jnp.int32, (BN, K), 1)
    idx = jnp.min(jnp.where(d == dmin, iota, K), axis=1)
    idx_ref[...] = idx[:, None]
    enc = (iota == idx[:, None]).astype(jnp.float32)
    enc_ref[...] = enc
    hist_ref[...] += jnp.sum(enc, axis=0, keepdims=True)
    acc_ref[0, 0] += jnp.sum(dmin)

    @pl.when(i == NB - 1)
    def _fin():
        p = hist_ref[...] * (1.0 / N)
        s = jnp.sum(p * jnp.log(p + 1e-10))
        scal_ref[0, 0] = (1.0 + BETA) * (acc_ref[0, 0] / (N * D))
        scal_ref[0, 1] = jnp.exp(-s)


_tc_call = pl.pallas_call(
    _tc_body,
    grid=(NB,),
    in_specs=[
        pl.BlockSpec((BN, 1), lambda i: (i, 0)),
        pl.BlockSpec((1, K), lambda i: (0, 0)),
        pl.BlockSpec((BN, D), lambda i: (i, 0)),
        pl.BlockSpec((K, D), lambda i: (0, 0)),
    ],
    out_specs=[
        pl.BlockSpec((BN, 1), lambda i: (i, 0)),
        pl.BlockSpec((BN, K), lambda i: (i, 0)),
        pl.BlockSpec(memory_space=pltpu.SMEM),
    ],
    out_shape=[
        jax.ShapeDtypeStruct((N, 1), jnp.int32),
        jax.ShapeDtypeStruct((N, K), jnp.float32),
        jax.ShapeDtypeStruct((1, 2), jnp.float32),
    ],
    scratch_shapes=[
        pltpu.VMEM((1, K), jnp.float32),
        pltpu.SMEM((1, 1), jnp.float32),
    ],
    compiler_params=pltpu.CompilerParams(
        dimension_semantics=("arbitrary",),
    ),
)


def _make_sc_gather():
    info = plsc.get_sparse_core_info()
    nc, ns = info.num_cores, info.num_subcores
    nw = nc * ns
    b_per_w = N // nw
    mesh = plsc.VectorSubcoreMesh(core_axis_name="c", subcore_axis_name="s")

    @functools.partial(
        pl.kernel, mesh=mesh,
        out_type=jax.ShapeDtypeStruct((N, D), jnp.float32),
        scratch_types=[
            pltpu.VMEM((b_per_w,), jnp.int32),
            pltpu.VMEM((b_per_w, D), jnp.float32),
            pltpu.SemaphoreType.DMA,
        ],
    )
    def gather_k(w_hbm, idx_hbm, out_hbm, idx_v, rows_v, sem):
        wid = lax.axis_index("s") * nc + lax.axis_index("c")
        base = wid * b_per_w
        pltpu.sync_copy(idx_hbm.at[pl.ds(base, b_per_w)], idx_v)
        pltpu.async_copy(w_hbm.at[idx_v], rows_v, sem).wait()
        pltpu.sync_copy(rows_v, out_hbm.at[pl.ds(base, b_per_w)])

    return gather_k


def kernel(inputs, W):
    x4 = jnp.transpose(inputs, (0, 2, 3, 1))
    input_shape = x4.shape
    flat = x4.reshape(-1, D)
    sx = jnp.sum(flat ** 2, axis=1, keepdims=True)
    sw = jnp.sum(W ** 2, axis=1).reshape(1, K)
    idx2, encodings, scal = _tc_call(sx, sw, flat, W)
    idx = idx2.reshape(N)
    qflat = _make_sc_gather()(W, idx)
    quantized_st = flat + (qflat - flat)
    quantized_st = jnp.transpose(quantized_st.reshape(input_shape), (0, 3, 1, 2))
    loss = scal[0, 0]
    perplexity = scal[0, 1]
    return (loss, quantized_st, perplexity, encodings)
